# baseline (speedup 1.0000x reference)
import functools

import jax
import jax.numpy as jnp
from jax import lax
from jax.experimental import pallas as pl
from jax.experimental.pallas import tpu as pltpu
from jax.experimental.pallas import tpu_sc as plsc

_B, _C, _T, _H, _W = 8, 3, 32, 224, 224
_NF = 4
_STRIDE = 8
_NBLK = _B * _C * _NF
_NC = 2
_BLK_PER_SCS = _NBLK // _NC
_NBUF = 8


def _sc_frame_gather(x3):
    mesh = plsc.ScalarSubcoreMesh(axis_name="c", num_cores=_NC)

    @functools.partial(
        pl.kernel,
        mesh=mesh,
        out_type=jax.ShapeDtypeStruct((_NBLK, _H, _W), jnp.float32),
        scratch_types=(
            [pltpu.VMEM_SHARED((_H, _W), jnp.float32) for _ in range(_NBUF)]
            + [pltpu.SemaphoreType.DMA for _ in range(2 * _NBUF)]
        ),
    )
    def k(x_hbm, out_hbm, *scratch):
        bufs = scratch[:_NBUF]
        sis = scratch[_NBUF:2 * _NBUF]
        sos = scratch[2 * _NBUF:3 * _NBUF]
        cid = lax.axis_index("c")

        def offs(i):
            g = cid * _BLK_PER_SCS + i
            bc = g // _NF
            j = g % _NF
            return bc * _T + _STRIDE * j, g

        gathers = [None] * _NBUF
        scatters = [None] * _NBUF
        for i in range(_BLK_PER_SCS):
            s = i % _NBUF
            src, _ = offs(i)
            if scatters[s] is not None:
                scatters[s].wait()
            gathers[s] = pltpu.async_copy(x_hbm.at[src], bufs[s], sis[s])
            if i >= 1:
                p = (i - 1) % _NBUF
                gathers[p].wait()
                _, pg = offs(i - 1)
                scatters[p] = pltpu.async_copy(bufs[p], out_hbm.at[pg], sos[p])
        p = (_BLK_PER_SCS - 1) % _NBUF
        gathers[p].wait()
        _, lg = offs(_BLK_PER_SCS - 1)
        scatters[p] = pltpu.async_copy(bufs[p], out_hbm.at[lg], sos[p])
        for s in range(_NBUF):
            if scatters[s] is not None:
                scatters[s].wait()

    return k(x3)


def kernel(x):
    x3 = x.reshape(_B * _C * _T, _H, _W)
    out = _sc_frame_gather(x3)
    return out.reshape(_B, _C, _NF, _H, _W)


# speedup vs baseline: 1.3570x; 1.3570x over previous
import functools

import jax
import jax.numpy as jnp
from jax import lax
from jax.experimental import pallas as pl
from jax.experimental.pallas import tpu as pltpu
from jax.experimental.pallas import tpu_sc as plsc

_B, _C, _T, _H, _W = 8, 3, 32, 224, 224
_NF = 4
_STRIDE = 8
_NBLK = _B * _C * _NF
_NC = 2
_NS = 16
_NW = _NC * _NS
_BLK_PER_W = _NBLK // _NW


def _sc_frame_gather(x3):
    mesh = plsc.VectorSubcoreMesh(core_axis_name="c", subcore_axis_name="s")

    @functools.partial(
        pl.kernel,
        mesh=mesh,
        out_type=jax.ShapeDtypeStruct((_NBLK, _H, _W), jnp.float32),
        scratch_types=[
            pltpu.VMEM((_H, _W), jnp.float32),
            pltpu.VMEM((_H, _W), jnp.float32),
            pltpu.SemaphoreType.DMA,
            pltpu.SemaphoreType.DMA,
            pltpu.SemaphoreType.DMA,
            pltpu.SemaphoreType.DMA,
        ],
    )
    def k(x_hbm, out_hbm, buf0, buf1, si0, si1, so0, so1):
        wid = lax.axis_index("c") * _NS + lax.axis_index("s")
        bufs = (buf0, buf1)
        sis = (si0, si1)
        sos = (so0, so1)

        def offs(kk):
            g = wid * _BLK_PER_W + kk
            bc = g // _NF
            j = g % _NF
            return bc * _T + _STRIDE * j, g

        gathers = [None, None]
        scatters = [None, None]
        for kk in range(_BLK_PER_W):
            s = kk % 2
            src, _ = offs(kk)
            if scatters[s] is not None:
                scatters[s].wait()
            gathers[s] = pltpu.async_copy(x_hbm.at[src], bufs[s], sis[s])
            if kk >= 1:
                p = (kk - 1) % 2
                gathers[p].wait()
                _, pdst = offs(kk - 1)
                scatters[p] = pltpu.async_copy(
                    bufs[p], out_hbm.at[pdst], sos[p]
                )
        last = (_BLK_PER_W - 1) % 2
        gathers[last].wait()
        _, ldst = offs(_BLK_PER_W - 1)
        scatters[last] = pltpu.async_copy(
            bufs[last], out_hbm.at[ldst], sos[last]
        )
        for s in range(2):
            if scatters[s] is not None:
                scatters[s].wait()

    return k(x3)


def kernel(x):
    x3 = x.reshape(_B * _C * _T, _H, _W)
    out = _sc_frame_gather(x3)
    return out.reshape(_B, _C, _NF, _H, _W)
